# baseline (device time: 135880 ns/iter reference)
import jax
import jax.numpy as jnp
from jax import lax
from jax.experimental import pallas as pl
from jax.experimental.pallas import tpu as pltpu

N = 4096
M_BLOCK = 4096
M_SUB = 1024
CHUNKS = (64, 64, 128, 128, 128, 128, 128, 128, 64, 64)
OFFS = tuple(sum(CHUNKS[:i]) for i in range(len(CHUNKS)))
NC = len(CHUNKS)
CH_MAX = max(CHUNKS)
EPS = 1e-6
QSCALE = 5.0 / 127.0
QSCALE_INV = 127.0 / 5.0
XSCALE = 5.5 / 127.0
XSCALE_INV = 127.0 / 5.5

CH2 = 512


def _comm_kernel(partial, gamma2):

    def body(partial_ref, gamma_ref, myb_ref, gath_ref,
             recv_x, local_buf, norm_q, my_f32, xload_buf, send_bf,
             sem_sendx, sem_recvx, sem_loc, sem_store, sem_xload,
             send2, recv2, sendF1, sendF2, recvF1, recvF2):
        x = lax.axis_index("x")
        y = lax.axis_index("y")
        z = lax.axis_index("z")
        q = 2 * y + z
        peer_x = 1 - x
        q_yn = 2 * (1 - y) + z
        q_zn = 2 * y + (1 - z)
        s_yn = lax.rem(q_yn - q + 3, 4)
        s_zn = lax.rem(q_zn - q + 3, 4)
        s_dg = lax.rem((2 * (1 - y) + (1 - z)) - q + 3, 4)

        barrier_sem = pltpu.get_barrier_semaphore()
        for dev in [(peer_x, y, z), (x, 1 - y, z), (x, y, 1 - z)]:
            pl.semaphore_signal(barrier_sem, inc=1, device_id=dev,
                                device_id_type=pl.DeviceIdType.MESH)
        pl.semaphore_wait(barrier_sem, 3)

        src_base = M_BLOCK * peer_x + M_SUB * q
        xloads = {}
        rdmax = {}

        def start_xload(c):
            ch, off = CHUNKS[c], OFFS[c]
            xloads[c] = pltpu.make_async_copy(
                partial_ref.at[0, pl.ds(src_base + off, ch), :],
                xload_buf.at[c % 3, pl.ds(0, ch), :], sem_xload.at[c % 3])
            xloads[c].start()

        def send_x(c):
            ch, off = CHUNKS[c], OFFS[c]
            xloads[c].wait()
            if c >= 3:
                rdmax[c - 3].wait_send()
            send_bf[c % 3, pl.ds(0, ch), :] = jnp.clip(
                jnp.round(xload_buf[c % 3, pl.ds(0, ch), :] * XSCALE_INV),
                -127.0, 127.0).astype(jnp.int8)
            if c + 2 < NC:
                start_xload(c + 2)
            r = pltpu.make_async_remote_copy(
                src_ref=send_bf.at[c % 3, pl.ds(0, ch), :],
                dst_ref=recv_x.at[pl.ds(off, ch), :],
                send_sem=sem_sendx.at[c % 3],
                recv_sem=sem_recvx.at[c],
                device_id=(peer_x, y, z),
                device_id_type=pl.DeviceIdType.MESH,
            )
            r.start()
            rdmax[c] = r

        PRO = 3
        start_xload(0)
        start_xload(1)
        for c in range(min(PRO, NC)):
            send_x(c)

        local_base = M_BLOCK * x + M_SUB * q
        loads = {}
        loads[0] = pltpu.make_async_copy(
            partial_ref.at[0, pl.ds(local_base, CHUNKS[0]), :],
            local_buf.at[0, pl.ds(0, CHUNKS[0]), :], sem_loc.at[0])
        loads[0].start()

        s_fwd_y = lax.rem(q_zn - q_yn + 3, 4)
        s_fwd_z = lax.rem(q_yn - q_zn + 3, 4)
        fw1 = {}
        fw2 = {}

        def process_forward(f):
            chf, offf = CHUNKS[f], OFFS[f]
            h = chf // 2
            rz = pltpu.make_async_remote_copy(
                src_ref=norm_q.at[0, pl.ds(0, chf), :],
                dst_ref=gath_ref.at[s_zn, pl.ds(offf, chf), :],
                send_sem=send2.at[0, f],
                recv_sem=recv2.at[s_zn, f],
                device_id=(x, y, z),
                device_id_type=pl.DeviceIdType.MESH,
            )
            rz.wait_recv()
            fw1[f] = pltpu.make_async_remote_copy(
                src_ref=gath_ref.at[s_zn, pl.ds(offf, h), :],
                dst_ref=gath_ref.at[s_fwd_y, pl.ds(offf, h), :],
                send_sem=sendF1.at[f],
                recv_sem=recvF1.at[f],
                device_id=(x, 1 - y, z),
                device_id_type=pl.DeviceIdType.MESH,
            )
            fw1[f].start()
            ry = pltpu.make_async_remote_copy(
                src_ref=norm_q.at[0, pl.ds(0, chf), :],
                dst_ref=gath_ref.at[s_yn, pl.ds(offf, chf), :],
                send_sem=send2.at[0, f],
                recv_sem=recv2.at[s_yn, f],
                device_id=(x, y, z),
                device_id_type=pl.DeviceIdType.MESH,
            )
            ry.wait_recv()
            fw2[f] = pltpu.make_async_remote_copy(
                src_ref=gath_ref.at[s_yn, pl.ds(offf + h, h), :],
                dst_ref=gath_ref.at[s_fwd_z, pl.ds(offf + h, h), :],
                send_sem=sendF2.at[f],
                recv_sem=recvF2.at[f],
                device_id=(x, y, 1 - z),
                device_id_type=pl.DeviceIdType.MESH,
            )
            fw2[f].start()

        sends2 = {}
        stores = {}
        for c in range(NC):
            ch, off = CHUNKS[c], OFFS[c]
            if c + PRO < NC:
                send_x(c + PRO)
            if c >= 1:
                process_forward(c - 1)
            if c + 1 < NC:
                chn, offn = CHUNKS[c + 1], OFFS[c + 1]
                loads[c + 1] = pltpu.make_async_copy(
                    partial_ref.at[0, pl.ds(local_base + offn, chn), :],
                    local_buf.at[(c + 1) % 2, pl.ds(0, chn), :],
                    sem_loc.at[(c + 1) % 2])
                loads[c + 1].start()
            rdmax[c].wait_recv()
            loads[c].wait()
            if c >= 3:
                for r in sends2[c - 3]:
                    r.wait_send()
                stores[c - 3].wait()
            ysum = (local_buf[c % 2, pl.ds(0, ch), :]
                    + recv_x[pl.ds(off, ch), :].astype(jnp.float32) * XSCALE)
            ms = jnp.mean(ysum * ysum, axis=-1, keepdims=True)
            normed = ysum * lax.rsqrt(ms + EPS)
            norm_q[c % 3, pl.ds(0, ch), :] = jnp.clip(
                jnp.round(normed * QSCALE_INV), -127.0, 127.0
            ).astype(jnp.int8)
            my_f32[c % 3, pl.ds(0, ch), :] = normed * gamma_ref[...]
            stores[c] = pltpu.make_async_copy(
                my_f32.at[c % 3, pl.ds(0, ch), :],
                myb_ref.at[pl.ds(off, ch), :], sem_store.at[c % 3])
            stores[c].start()
            sends2[c] = []
            for idx, (dev, q_r) in enumerate(
                    [((x, 1 - y, z), q_yn), ((x, y, 1 - z), q_zn)]):
                s = lax.rem(q - q_r + 3, 4)
                r = pltpu.make_async_remote_copy(
                    src_ref=norm_q.at[c % 3, pl.ds(0, ch), :],
                    dst_ref=gath_ref.at[s, pl.ds(off, ch), :],
                    send_sem=send2.at[idx, c],
                    recv_sem=recv2.at[s, c],
                    device_id=dev,
                    device_id_type=pl.DeviceIdType.MESH,
                )
                r.start()
                sends2[c].append(r)

        process_forward(NC - 1)
        for c in range(max(NC - 3, 0), NC):
            for r in sends2[c]:
                r.wait_send()
            stores[c].wait()
            rdmax[c].wait_send()
        for f in range(NC):
            fw1[f].wait_send()
            fw2[f].wait_send()
        for f in range(NC):
            chf, offf = CHUNKS[f], OFFS[f]
            h = chf // 2
            r1 = pltpu.make_async_remote_copy(
                src_ref=norm_q.at[0, pl.ds(0, h), :],
                dst_ref=gath_ref.at[s_dg, pl.ds(offf, h), :],
                send_sem=sendF1.at[f],
                recv_sem=recvF1.at[f],
                device_id=(x, y, z),
                device_id_type=pl.DeviceIdType.MESH,
            )
            r1.wait_recv()
            r2 = pltpu.make_async_remote_copy(
                src_ref=norm_q.at[0, pl.ds(0, h), :],
                dst_ref=gath_ref.at[s_dg, pl.ds(offf + h, h), :],
                send_sem=sendF2.at[f],
                recv_sem=recvF2.at[f],
                device_id=(x, y, z),
                device_id_type=pl.DeviceIdType.MESH,
            )
            r2.wait_recv()

    return pl.pallas_call(
        body,
        out_shape=(
            jax.ShapeDtypeStruct((M_SUB, N), jnp.float32),
            jax.ShapeDtypeStruct((3, M_SUB, N), jnp.int8),
        ),
        in_specs=[pl.BlockSpec(memory_space=pl.ANY),
                  pl.BlockSpec(memory_space=pltpu.VMEM)],
        out_specs=(pl.BlockSpec(memory_space=pl.ANY),
                   pl.BlockSpec(memory_space=pl.ANY)),
        scratch_shapes=[
            pltpu.VMEM((M_SUB, N), jnp.int8),
            pltpu.VMEM((2, CH_MAX, N), jnp.float32),
            pltpu.VMEM((3, CH_MAX, N), jnp.int8),
            pltpu.VMEM((3, CH_MAX, N), jnp.float32),
            pltpu.VMEM((3, CH_MAX, N), jnp.float32),
            pltpu.VMEM((3, CH_MAX, N), jnp.int8),
            pltpu.SemaphoreType.DMA((3,)),
            pltpu.SemaphoreType.DMA((NC,)),
            pltpu.SemaphoreType.DMA((2,)),
            pltpu.SemaphoreType.DMA((3,)),
            pltpu.SemaphoreType.DMA((3,)),
            pltpu.SemaphoreType.DMA((2, NC)),
            pltpu.SemaphoreType.DMA((3, NC)),
            pltpu.SemaphoreType.DMA((NC,)),
            pltpu.SemaphoreType.DMA((NC,)),
            pltpu.SemaphoreType.DMA((NC,)),
            pltpu.SemaphoreType.DMA((NC,)),
        ],
        compiler_params=pltpu.CompilerParams(
            collective_id=0, vmem_limit_bytes=100 * 1024 * 1024),
    )(partial, gamma2)


def _assemble_kernel(myb, gath, gamma2):

    def body(myb_ref, gath_ref, gamma_ref, out_ref,
             i8_buf, f32_buf, my_buf, sem_i8, sem_out, sem_my, sem_my2):
        x = lax.axis_index("x")
        y = lax.axis_index("y")
        z = lax.axis_index("z")
        q = 2 * y + z

        my_ld = pltpu.make_async_copy(myb_ref, my_buf, sem_my)
        my_ld.start()

        n_sub = M_SUB // CH2
        total = 3 * n_sub
        loads = {}

        def start_load(i):
            s, k = divmod(i, n_sub)
            loads[i] = pltpu.make_async_copy(
                gath_ref.at[s, pl.ds(k * CH2, CH2), :],
                i8_buf.at[i % 2], sem_i8.at[i % 2])
            loads[i].start()

        start_load(0)
        start_load(1)
        stores = {}
        for i in range(total):
            s, k = divmod(i, n_sub)
            q_s = lax.rem(q + s + 1, 4)
            loads[i].wait()
            if i >= 2:
                stores[i - 2].wait()
            f32_buf[i % 2] = (i8_buf[i % 2].astype(jnp.float32)
                              * QSCALE * gamma_ref[...])
            stores[i] = pltpu.make_async_copy(
                f32_buf.at[i % 2],
                out_ref.at[pl.ds(M_SUB * q_s + k * CH2, CH2), :],
                sem_out.at[i % 2])
            stores[i].start()
            if i + 2 < total:
                start_load(i + 2)
        my_ld.wait()
        my_st = pltpu.make_async_copy(
            my_buf, out_ref.at[pl.ds(M_SUB * q, M_SUB), :], sem_my2)
        my_st.start()
        stores[total - 2].wait()
        stores[total - 1].wait()
        my_st.wait()

    return pl.pallas_call(
        body,
        out_shape=jax.ShapeDtypeStruct((M_BLOCK, N), jnp.float32),
        in_specs=[pl.BlockSpec(memory_space=pl.ANY),
                  pl.BlockSpec(memory_space=pl.ANY),
                  pl.BlockSpec(memory_space=pltpu.VMEM)],
        out_specs=pl.BlockSpec(memory_space=pl.ANY),
        scratch_shapes=[
            pltpu.VMEM((2, CH2, N), jnp.int8),
            pltpu.VMEM((2, CH2, N), jnp.float32),
            pltpu.VMEM((M_SUB, N), jnp.float32),
            pltpu.SemaphoreType.DMA((2,)),
            pltpu.SemaphoreType.DMA((2,)),
            pltpu.SemaphoreType.DMA,
            pltpu.SemaphoreType.DMA,
        ],
        compiler_params=pltpu.CompilerParams(
            vmem_limit_bytes=100 * 1024 * 1024),
    )(myb, gath, gamma2)


def kernel(partial, gamma):
    gamma2 = gamma.reshape(1, N)
    myb, gath = _comm_kernel(partial, gamma2)
    return _assemble_kernel(myb, gath, gamma2)


# device time: 116423 ns/iter; 1.1671x vs baseline; 1.1671x over previous
import jax
import jax.numpy as jnp
from jax import lax
from jax.experimental import pallas as pl
from jax.experimental.pallas import tpu as pltpu

N = 4096
M_BLOCK = 4096
M_SUB = 1024
CHUNKS = (64, 64, 128, 128, 128, 128, 128, 128, 64, 64)
OFFS = tuple(sum(CHUNKS[:i]) for i in range(len(CHUNKS)))
NC = len(CHUNKS)
CH_MAX = max(CHUNKS)
EPS = 1e-6
QSCALE = 5.0 / 127.0
QSCALE_INV = 127.0 / 5.0
XSCALE = 5.5 / 127.0
XSCALE_INV = 127.0 / 5.5

CH2 = 512


def _comm_kernel(partial, gamma2):

    def body(partial_ref, gamma_ref, myb_ref, gath_ref,
             recv_x, local_buf, norm_q, my_f32, xload_buf, send_bf,
             sem_sendx, sem_recvx, sem_loc, sem_store, sem_xload,
             send2, recv2, sendF1, sendF2, recvF1, recvF2):
        x = lax.axis_index("x")
        y = lax.axis_index("y")
        z = lax.axis_index("z")
        q = 2 * y + z
        peer_x = 1 - x
        q_yn = 2 * (1 - y) + z
        q_zn = 2 * y + (1 - z)
        s_yn = lax.rem(q_yn - q + 3, 4)
        s_zn = lax.rem(q_zn - q + 3, 4)
        s_dg = lax.rem((2 * (1 - y) + (1 - z)) - q + 3, 4)

        barrier_sem = pltpu.get_barrier_semaphore()
        for dev in [(peer_x, y, z), (x, 1 - y, z), (x, y, 1 - z)]:
            pl.semaphore_signal(barrier_sem, inc=1, device_id=dev,
                                device_id_type=pl.DeviceIdType.MESH)
        pl.semaphore_wait(barrier_sem, 3)

        src_base = M_BLOCK * peer_x + M_SUB * q
        xloads = {}
        rdmax = {}

        def start_xload(c):
            ch, off = CHUNKS[c], OFFS[c]
            xloads[c] = pltpu.make_async_copy(
                partial_ref.at[0, pl.ds(src_base + off, ch), :],
                xload_buf.at[c % 3, pl.ds(0, ch), :], sem_xload.at[c % 3])
            xloads[c].start()

        def send_x(c):
            ch, off = CHUNKS[c], OFFS[c]
            xloads[c].wait()
            if c >= 3:
                rdmax[c - 3].wait_send()
            send_bf[c % 3, pl.ds(0, ch), :] = jnp.clip(
                jnp.round(xload_buf[c % 3, pl.ds(0, ch), :] * XSCALE_INV),
                -127.0, 127.0).astype(jnp.int8)
            if c + 2 < NC:
                start_xload(c + 2)
            r = pltpu.make_async_remote_copy(
                src_ref=send_bf.at[c % 3, pl.ds(0, ch), :],
                dst_ref=recv_x.at[pl.ds(off, ch), :],
                send_sem=sem_sendx.at[c % 3],
                recv_sem=sem_recvx.at[c],
                device_id=(peer_x, y, z),
                device_id_type=pl.DeviceIdType.MESH,
            )
            r.start()
            rdmax[c] = r

        PRO = 3
        start_xload(0)
        start_xload(1)
        for c in range(min(PRO, NC)):
            send_x(c)

        local_base = M_BLOCK * x + M_SUB * q
        loads = {}
        loads[0] = pltpu.make_async_copy(
            partial_ref.at[0, pl.ds(local_base, CHUNKS[0]), :],
            local_buf.at[0, pl.ds(0, CHUNKS[0]), :], sem_loc.at[0])
        loads[0].start()

        s_fwd_y = lax.rem(q_zn - q_yn + 3, 4)
        s_fwd_z = lax.rem(q_yn - q_zn + 3, 4)
        fw1 = {}
        fw2 = {}

        def process_forward(f):
            chf, offf = CHUNKS[f], OFFS[f]
            h = chf // 2
            rz = pltpu.make_async_remote_copy(
                src_ref=norm_q.at[0, pl.ds(0, chf), :],
                dst_ref=gath_ref.at[s_zn, pl.ds(offf, chf), :],
                send_sem=send2.at[0, f],
                recv_sem=recv2.at[s_zn, f],
                device_id=(x, y, z),
                device_id_type=pl.DeviceIdType.MESH,
            )
            rz.wait_recv()
            fw1[f] = pltpu.make_async_remote_copy(
                src_ref=gath_ref.at[s_zn, pl.ds(offf, h), :],
                dst_ref=gath_ref.at[s_fwd_y, pl.ds(offf, h), :],
                send_sem=sendF1.at[f],
                recv_sem=recvF1.at[f],
                device_id=(x, 1 - y, z),
                device_id_type=pl.DeviceIdType.MESH,
            )
            fw1[f].start()
            ry = pltpu.make_async_remote_copy(
                src_ref=norm_q.at[0, pl.ds(0, chf), :],
                dst_ref=gath_ref.at[s_yn, pl.ds(offf, chf), :],
                send_sem=send2.at[0, f],
                recv_sem=recv2.at[s_yn, f],
                device_id=(x, y, z),
                device_id_type=pl.DeviceIdType.MESH,
            )
            ry.wait_recv()
            fw2[f] = pltpu.make_async_remote_copy(
                src_ref=gath_ref.at[s_yn, pl.ds(offf + h, h), :],
                dst_ref=gath_ref.at[s_fwd_z, pl.ds(offf + h, h), :],
                send_sem=sendF2.at[f],
                recv_sem=recvF2.at[f],
                device_id=(x, y, 1 - z),
                device_id_type=pl.DeviceIdType.MESH,
            )
            fw2[f].start()

        sends2 = {}
        stores = {}
        for c in range(NC):
            ch, off = CHUNKS[c], OFFS[c]
            if c + PRO < NC:
                send_x(c + PRO)
            if c >= 3:
                process_forward(c - 3)
            if c + 1 < NC:
                chn, offn = CHUNKS[c + 1], OFFS[c + 1]
                loads[c + 1] = pltpu.make_async_copy(
                    partial_ref.at[0, pl.ds(local_base + offn, chn), :],
                    local_buf.at[(c + 1) % 2, pl.ds(0, chn), :],
                    sem_loc.at[(c + 1) % 2])
                loads[c + 1].start()
            rdmax[c].wait_recv()
            loads[c].wait()
            if c >= 3:
                for r in sends2[c - 3]:
                    r.wait_send()
                stores[c - 3].wait()
            ysum = (local_buf[c % 2, pl.ds(0, ch), :]
                    + recv_x[pl.ds(off, ch), :].astype(jnp.float32) * XSCALE)
            ms = jnp.mean(ysum * ysum, axis=-1, keepdims=True)
            normed = ysum * lax.rsqrt(ms + EPS)
            norm_q[c % 3, pl.ds(0, ch), :] = jnp.clip(
                jnp.round(normed * QSCALE_INV), -127.0, 127.0
            ).astype(jnp.int8)
            my_f32[c % 3, pl.ds(0, ch), :] = normed * gamma_ref[...]
            stores[c] = pltpu.make_async_copy(
                my_f32.at[c % 3, pl.ds(0, ch), :],
                myb_ref.at[pl.ds(off, ch), :], sem_store.at[c % 3])
            stores[c].start()
            sends2[c] = []
            for idx, (dev, q_r) in enumerate(
                    [((x, 1 - y, z), q_yn), ((x, y, 1 - z), q_zn)]):
                s = lax.rem(q - q_r + 3, 4)
                r = pltpu.make_async_remote_copy(
                    src_ref=norm_q.at[c % 3, pl.ds(0, ch), :],
                    dst_ref=gath_ref.at[s, pl.ds(off, ch), :],
                    send_sem=send2.at[idx, c],
                    recv_sem=recv2.at[s, c],
                    device_id=dev,
                    device_id_type=pl.DeviceIdType.MESH,
                )
                r.start()
                sends2[c].append(r)

        for f in range(max(NC - 3, 0), NC):
            process_forward(f)
        for c in range(max(NC - 3, 0), NC):
            for r in sends2[c]:
                r.wait_send()
            stores[c].wait()
            rdmax[c].wait_send()
        for f in range(NC):
            fw1[f].wait_send()
            fw2[f].wait_send()
        for f in range(NC):
            chf, offf = CHUNKS[f], OFFS[f]
            h = chf // 2
            r1 = pltpu.make_async_remote_copy(
                src_ref=norm_q.at[0, pl.ds(0, h), :],
                dst_ref=gath_ref.at[s_dg, pl.ds(offf, h), :],
                send_sem=sendF1.at[f],
                recv_sem=recvF1.at[f],
                device_id=(x, y, z),
                device_id_type=pl.DeviceIdType.MESH,
            )
            r1.wait_recv()
            r2 = pltpu.make_async_remote_copy(
                src_ref=norm_q.at[0, pl.ds(0, h), :],
                dst_ref=gath_ref.at[s_dg, pl.ds(offf + h, h), :],
                send_sem=sendF2.at[f],
                recv_sem=recvF2.at[f],
                device_id=(x, y, z),
                device_id_type=pl.DeviceIdType.MESH,
            )
            r2.wait_recv()

    return pl.pallas_call(
        body,
        out_shape=(
            jax.ShapeDtypeStruct((M_SUB, N), jnp.float32),
            jax.ShapeDtypeStruct((3, M_SUB, N), jnp.int8),
        ),
        in_specs=[pl.BlockSpec(memory_space=pl.ANY),
                  pl.BlockSpec(memory_space=pltpu.VMEM)],
        out_specs=(pl.BlockSpec(memory_space=pl.ANY),
                   pl.BlockSpec(memory_space=pl.ANY)),
        scratch_shapes=[
            pltpu.VMEM((M_SUB, N), jnp.int8),
            pltpu.VMEM((2, CH_MAX, N), jnp.float32),
            pltpu.VMEM((3, CH_MAX, N), jnp.int8),
            pltpu.VMEM((3, CH_MAX, N), jnp.float32),
            pltpu.VMEM((3, CH_MAX, N), jnp.float32),
            pltpu.VMEM((3, CH_MAX, N), jnp.int8),
            pltpu.SemaphoreType.DMA((3,)),
            pltpu.SemaphoreType.DMA((NC,)),
            pltpu.SemaphoreType.DMA((2,)),
            pltpu.SemaphoreType.DMA((3,)),
            pltpu.SemaphoreType.DMA((3,)),
            pltpu.SemaphoreType.DMA((2, NC)),
            pltpu.SemaphoreType.DMA((3, NC)),
            pltpu.SemaphoreType.DMA((NC,)),
            pltpu.SemaphoreType.DMA((NC,)),
            pltpu.SemaphoreType.DMA((NC,)),
            pltpu.SemaphoreType.DMA((NC,)),
        ],
        compiler_params=pltpu.CompilerParams(
            collective_id=0, vmem_limit_bytes=100 * 1024 * 1024),
    )(partial, gamma2)


def _assemble_kernel(myb, gath, gamma2):

    def body(myb_ref, gath_ref, gamma_ref, out_ref,
             i8_buf, f32_buf, my_buf, sem_i8, sem_out, sem_my, sem_my2):
        x = lax.axis_index("x")
        y = lax.axis_index("y")
        z = lax.axis_index("z")
        q = 2 * y + z

        my_ld = pltpu.make_async_copy(myb_ref, my_buf, sem_my)
        my_ld.start()

        n_sub = M_SUB // CH2
        total = 3 * n_sub
        loads = {}

        def start_load(i):
            s, k = divmod(i, n_sub)
            loads[i] = pltpu.make_async_copy(
                gath_ref.at[s, pl.ds(k * CH2, CH2), :],
                i8_buf.at[i % 2], sem_i8.at[i % 2])
            loads[i].start()

        start_load(0)
        start_load(1)
        stores = {}
        for i in range(total):
            s, k = divmod(i, n_sub)
            q_s = lax.rem(q + s + 1, 4)
            loads[i].wait()
            if i >= 2:
                stores[i - 2].wait()
            f32_buf[i % 2] = (i8_buf[i % 2].astype(jnp.float32)
                              * QSCALE * gamma_ref[...])
            stores[i] = pltpu.make_async_copy(
                f32_buf.at[i % 2],
                out_ref.at[pl.ds(M_SUB * q_s + k * CH2, CH2), :],
                sem_out.at[i % 2])
            stores[i].start()
            if i + 2 < total:
                start_load(i + 2)
        my_ld.wait()
        my_st = pltpu.make_async_copy(
            my_buf, out_ref.at[pl.ds(M_SUB * q, M_SUB), :], sem_my2)
        my_st.start()
        stores[total - 2].wait()
        stores[total - 1].wait()
        my_st.wait()

    return pl.pallas_call(
        body,
        out_shape=jax.ShapeDtypeStruct((M_BLOCK, N), jnp.float32),
        in_specs=[pl.BlockSpec(memory_space=pl.ANY),
                  pl.BlockSpec(memory_space=pl.ANY),
                  pl.BlockSpec(memory_space=pltpu.VMEM)],
        out_specs=pl.BlockSpec(memory_space=pl.ANY),
        scratch_shapes=[
            pltpu.VMEM((2, CH2, N), jnp.int8),
            pltpu.VMEM((2, CH2, N), jnp.float32),
            pltpu.VMEM((M_SUB, N), jnp.float32),
            pltpu.SemaphoreType.DMA((2,)),
            pltpu.SemaphoreType.DMA((2,)),
            pltpu.SemaphoreType.DMA,
            pltpu.SemaphoreType.DMA,
        ],
        compiler_params=pltpu.CompilerParams(
            vmem_limit_bytes=100 * 1024 * 1024),
    )(myb, gath, gamma2)


def kernel(partial, gamma):
    gamma2 = gamma.reshape(1, N)
    myb, gath = _comm_kernel(partial, gamma2)
    return _assemble_kernel(myb, gath, gamma2)


# device time: 113468 ns/iter; 1.1975x vs baseline; 1.0260x over previous
import jax
import jax.numpy as jnp
from jax import lax
from jax.experimental import pallas as pl
from jax.experimental.pallas import tpu as pltpu

N = 4096
M_BLOCK = 4096
M_SUB = 1024
CHUNKS = (64, 64, 128, 128, 128, 128, 128, 128, 64, 64)
OFFS = tuple(sum(CHUNKS[:i]) for i in range(len(CHUNKS)))
NC = len(CHUNKS)
CH_MAX = max(CHUNKS)
EPS = 1e-6
QSCALE = 5.0 / 127.0
QSCALE_INV = 127.0 / 5.0
XSCALE = 5.5 / 127.0
XSCALE_INV = 127.0 / 5.5

CH2 = 512


def _comm_kernel(partial, gamma2):

    def body(partial_ref, gamma_ref, gath_ref,
             recv_x, local_buf, norm_q, xload_buf, send_bf,
             sem_sendx, sem_recvx, sem_loc, sem_store, sem_xload,
             send2, recv2, sendF1, sendF2, recvF1, recvF2):
        x = lax.axis_index("x")
        y = lax.axis_index("y")
        z = lax.axis_index("z")
        q = 2 * y + z
        peer_x = 1 - x
        q_yn = 2 * (1 - y) + z
        q_zn = 2 * y + (1 - z)
        s_yn = lax.rem(q_yn - q + 3, 4)
        s_zn = lax.rem(q_zn - q + 3, 4)
        s_dg = lax.rem((2 * (1 - y) + (1 - z)) - q + 3, 4)

        barrier_sem = pltpu.get_barrier_semaphore()
        for dev in [(peer_x, y, z), (x, 1 - y, z), (x, y, 1 - z)]:
            pl.semaphore_signal(barrier_sem, inc=1, device_id=dev,
                                device_id_type=pl.DeviceIdType.MESH)
        pl.semaphore_wait(barrier_sem, 3)

        src_base = M_BLOCK * peer_x + M_SUB * q
        xloads = {}
        rdmax = {}

        def start_xload(c):
            ch, off = CHUNKS[c], OFFS[c]
            xloads[c] = pltpu.make_async_copy(
                partial_ref.at[0, pl.ds(src_base + off, ch), :],
                xload_buf.at[c % 3, pl.ds(0, ch), :], sem_xload.at[c % 3])
            xloads[c].start()

        def send_x(c):
            ch, off = CHUNKS[c], OFFS[c]
            xloads[c].wait()
            if c >= 3:
                rdmax[c - 3].wait_send()
            send_bf[c % 3, pl.ds(0, ch), :] = jnp.clip(
                jnp.round(xload_buf[c % 3, pl.ds(0, ch), :] * XSCALE_INV),
                -127.0, 127.0).astype(jnp.int8)
            if c + 2 < NC:
                start_xload(c + 2)
            r = pltpu.make_async_remote_copy(
                src_ref=send_bf.at[c % 3, pl.ds(0, ch), :],
                dst_ref=recv_x.at[pl.ds(off, ch), :],
                send_sem=sem_sendx.at[c % 3],
                recv_sem=sem_recvx.at[c],
                device_id=(peer_x, y, z),
                device_id_type=pl.DeviceIdType.MESH,
            )
            r.start()
            rdmax[c] = r

        PRO = 3
        start_xload(0)
        start_xload(1)
        for c in range(min(PRO, NC)):
            send_x(c)

        local_base = M_BLOCK * x + M_SUB * q
        loads = {}
        loads[0] = pltpu.make_async_copy(
            partial_ref.at[0, pl.ds(local_base, CHUNKS[0]), :],
            local_buf.at[0, pl.ds(0, CHUNKS[0]), :], sem_loc.at[0])
        loads[0].start()

        s_fwd_y = lax.rem(q_zn - q_yn + 3, 4)
        s_fwd_z = lax.rem(q_yn - q_zn + 3, 4)
        fw1 = {}
        fw2 = {}

        def process_forward(f):
            chf, offf = CHUNKS[f], OFFS[f]
            h = chf // 2
            rz = pltpu.make_async_remote_copy(
                src_ref=norm_q.at[0, pl.ds(0, chf), :],
                dst_ref=gath_ref.at[s_zn, pl.ds(offf, chf), :],
                send_sem=send2.at[0, f],
                recv_sem=recv2.at[s_zn, f],
                device_id=(x, y, z),
                device_id_type=pl.DeviceIdType.MESH,
            )
            rz.wait_recv()
            fw1[f] = pltpu.make_async_remote_copy(
                src_ref=gath_ref.at[s_zn, pl.ds(offf, h), :],
                dst_ref=gath_ref.at[s_fwd_y, pl.ds(offf, h), :],
                send_sem=sendF1.at[f],
                recv_sem=recvF1.at[f],
                device_id=(x, 1 - y, z),
                device_id_type=pl.DeviceIdType.MESH,
            )
            fw1[f].start()
            ry = pltpu.make_async_remote_copy(
                src_ref=norm_q.at[0, pl.ds(0, chf), :],
                dst_ref=gath_ref.at[s_yn, pl.ds(offf, chf), :],
                send_sem=send2.at[0, f],
                recv_sem=recv2.at[s_yn, f],
                device_id=(x, y, z),
                device_id_type=pl.DeviceIdType.MESH,
            )
            ry.wait_recv()
            fw2[f] = pltpu.make_async_remote_copy(
                src_ref=gath_ref.at[s_yn, pl.ds(offf + h, h), :],
                dst_ref=gath_ref.at[s_fwd_z, pl.ds(offf + h, h), :],
                send_sem=sendF2.at[f],
                recv_sem=recvF2.at[f],
                device_id=(x, y, 1 - z),
                device_id_type=pl.DeviceIdType.MESH,
            )
            fw2[f].start()

        sends2 = {}
        stores = {}
        for c in range(NC):
            ch, off = CHUNKS[c], OFFS[c]
            if c + PRO < NC:
                send_x(c + PRO)
            if c >= 3:
                process_forward(c - 3)
            if c + 1 < NC:
                chn, offn = CHUNKS[c + 1], OFFS[c + 1]
                loads[c + 1] = pltpu.make_async_copy(
                    partial_ref.at[0, pl.ds(local_base + offn, chn), :],
                    local_buf.at[(c + 1) % 2, pl.ds(0, chn), :],
                    sem_loc.at[(c + 1) % 2])
                loads[c + 1].start()
            rdmax[c].wait_recv()
            loads[c].wait()
            if c >= 3:
                for r in sends2[c - 3]:
                    r.wait_send()
                stores[c - 3].wait()
            ysum = (local_buf[c % 2, pl.ds(0, ch), :]
                    + recv_x[pl.ds(off, ch), :].astype(jnp.float32) * XSCALE)
            ms = jnp.mean(ysum * ysum, axis=-1, keepdims=True)
            normed = ysum * lax.rsqrt(ms + EPS)
            norm_q[c % 3, pl.ds(0, ch), :] = jnp.clip(
                jnp.round(normed * QSCALE_INV), -127.0, 127.0
            ).astype(jnp.int8)
            stores[c] = pltpu.make_async_copy(
                norm_q.at[c % 3, pl.ds(0, ch), :],
                gath_ref.at[3, pl.ds(off, ch), :], sem_store.at[c % 3])
            stores[c].start()
            sends2[c] = []
            for idx, (dev, q_r) in enumerate(
                    [((x, 1 - y, z), q_yn), ((x, y, 1 - z), q_zn)]):
                s = lax.rem(q - q_r + 3, 4)
                r = pltpu.make_async_remote_copy(
                    src_ref=norm_q.at[c % 3, pl.ds(0, ch), :],
                    dst_ref=gath_ref.at[s, pl.ds(off, ch), :],
                    send_sem=send2.at[idx, c],
                    recv_sem=recv2.at[s, c],
                    device_id=dev,
                    device_id_type=pl.DeviceIdType.MESH,
                )
                r.start()
                sends2[c].append(r)

        for f in range(max(NC - 3, 0), NC):
            process_forward(f)
        for c in range(max(NC - 3, 0), NC):
            for r in sends2[c]:
                r.wait_send()
            stores[c].wait()
            rdmax[c].wait_send()
        for f in range(NC):
            fw1[f].wait_send()
            fw2[f].wait_send()
        for f in range(NC):
            chf, offf = CHUNKS[f], OFFS[f]
            h = chf // 2
            r1 = pltpu.make_async_remote_copy(
                src_ref=norm_q.at[0, pl.ds(0, h), :],
                dst_ref=gath_ref.at[s_dg, pl.ds(offf, h), :],
                send_sem=sendF1.at[f],
                recv_sem=recvF1.at[f],
                device_id=(x, y, z),
                device_id_type=pl.DeviceIdType.MESH,
            )
            r1.wait_recv()
            r2 = pltpu.make_async_remote_copy(
                src_ref=norm_q.at[0, pl.ds(0, h), :],
                dst_ref=gath_ref.at[s_dg, pl.ds(offf + h, h), :],
                send_sem=sendF2.at[f],
                recv_sem=recvF2.at[f],
                device_id=(x, y, z),
                device_id_type=pl.DeviceIdType.MESH,
            )
            r2.wait_recv()

    return pl.pallas_call(
        body,
        out_shape=jax.ShapeDtypeStruct((4, M_SUB, N), jnp.int8),
        in_specs=[pl.BlockSpec(memory_space=pl.ANY),
                  pl.BlockSpec(memory_space=pltpu.VMEM)],
        out_specs=pl.BlockSpec(memory_space=pl.ANY),
        scratch_shapes=[
            pltpu.VMEM((M_SUB, N), jnp.int8),
            pltpu.VMEM((2, CH_MAX, N), jnp.float32),
            pltpu.VMEM((3, CH_MAX, N), jnp.int8),
            pltpu.VMEM((3, CH_MAX, N), jnp.float32),
            pltpu.VMEM((3, CH_MAX, N), jnp.int8),
            pltpu.SemaphoreType.DMA((3,)),
            pltpu.SemaphoreType.DMA((NC,)),
            pltpu.SemaphoreType.DMA((2,)),
            pltpu.SemaphoreType.DMA((3,)),
            pltpu.SemaphoreType.DMA((3,)),
            pltpu.SemaphoreType.DMA((2, NC)),
            pltpu.SemaphoreType.DMA((3, NC)),
            pltpu.SemaphoreType.DMA((NC,)),
            pltpu.SemaphoreType.DMA((NC,)),
            pltpu.SemaphoreType.DMA((NC,)),
            pltpu.SemaphoreType.DMA((NC,)),
        ],
        compiler_params=pltpu.CompilerParams(
            collective_id=0, vmem_limit_bytes=100 * 1024 * 1024),
    )(partial, gamma2)


def _assemble_kernel(gath, gamma2):

    def body(gath_ref, gamma_ref, out_ref,
             i8_buf, f32_buf, sem_i8, sem_out):
        x = lax.axis_index("x")
        y = lax.axis_index("y")
        z = lax.axis_index("z")
        q = 2 * y + z

        n_sub = M_SUB // CH2
        total = 4 * n_sub
        loads = {}

        def start_load(i):
            s, k = divmod(i, n_sub)
            loads[i] = pltpu.make_async_copy(
                gath_ref.at[s, pl.ds(k * CH2, CH2), :],
                i8_buf.at[i % 2], sem_i8.at[i % 2])
            loads[i].start()

        start_load(0)
        start_load(1)
        stores = {}
        for i in range(total):
            s, k = divmod(i, n_sub)
            q_s = lax.rem(q + s + 1, 4)
            loads[i].wait()
            if i >= 2:
                stores[i - 2].wait()
            f32_buf[i % 2] = (i8_buf[i % 2].astype(jnp.float32)
                              * QSCALE * gamma_ref[...])
            stores[i] = pltpu.make_async_copy(
                f32_buf.at[i % 2],
                out_ref.at[pl.ds(M_SUB * q_s + k * CH2, CH2), :],
                sem_out.at[i % 2])
            stores[i].start()
            if i + 2 < total:
                start_load(i + 2)
        stores[total - 2].wait()
        stores[total - 1].wait()

    return pl.pallas_call(
        body,
        out_shape=jax.ShapeDtypeStruct((M_BLOCK, N), jnp.float32),
        in_specs=[pl.BlockSpec(memory_space=pl.ANY),
                  pl.BlockSpec(memory_space=pltpu.VMEM)],
        out_specs=pl.BlockSpec(memory_space=pl.ANY),
        scratch_shapes=[
            pltpu.VMEM((2, CH2, N), jnp.int8),
            pltpu.VMEM((2, CH2, N), jnp.float32),
            pltpu.SemaphoreType.DMA((2,)),
            pltpu.SemaphoreType.DMA((2,)),
        ],
        compiler_params=pltpu.CompilerParams(
            vmem_limit_bytes=100 * 1024 * 1024),
    )(gath, gamma2)


def kernel(partial, gamma):
    gamma2 = gamma.reshape(1, N)
    gath = _comm_kernel(partial, gamma2)
    return _assemble_kernel(gath, gamma2)
